# speedup vs baseline: 1.6569x; 1.6569x over previous
import functools

import jax
import jax.numpy as jnp
from jax import lax
from jax.experimental import pallas as pl
from jax.experimental.pallas import tpu as pltpu

N_NODES = 10000
H = 256
G_GRAPHS = 64
LRELU = 0.01


def _lrelu(v):
    return jnp.where(v >= 0, v, LRELU * v)


def _elu(v):
    return jnp.where(v > 0, v, jnp.expm1(v))


def _mm_body(x_ref, w_ref, o_ref):
    o_ref[...] = jnp.dot(x_ref[...], w_ref[...],
                         preferred_element_type=jnp.float32)


def _pick_bm(m):
    for bm in (2048, 2000, 1024, 1000, 640, 512, 400, 256, 200, 128, 100, 64, 50, 8, 1):
        if m % bm == 0:
            return bm
    return m


def _matmul(x, w):
    m, k = x.shape
    n = w.shape[1]
    bm = _pick_bm(m)
    return pl.pallas_call(
        _mm_body,
        grid=(m // bm,),
        in_specs=[
            pl.BlockSpec((bm, k), lambda i: (i, 0)),
            pl.BlockSpec((k, n), lambda i: (0, 0)),
        ],
        out_specs=pl.BlockSpec((bm, n), lambda i: (i, 0)),
        out_shape=jax.ShapeDtypeStruct((m, n), jnp.float32),
    )(x, w)


def _gru(xin, hid, Wih, Whh, bih, bhh):
    gi = _matmul(xin, Wih.T) + bih
    gh = _matmul(hid, Whh.T) + bhh
    ir, iz, inn = jnp.split(gi, 3, axis=-1)
    hr, hz, hn = jnp.split(gh, 3, axis=-1)
    r = jax.nn.sigmoid(ir + hr)
    z = jax.nn.sigmoid(iz + hz)
    nn_ = jnp.tanh(inn + r * hn)
    return (1.0 - z) * nn_ + z * hid


def kernel(x, edge_index, edge_attr, batch, lin1_W, lin1_b, gate_W1, gate_W2,
           gate_att_l, gate_att_r, gate_bias, gru_Wih, gru_Whh, gru_bih,
           gru_bhh, atom_W, atom_att_src, atom_att_dst, atom_bias, mol_W,
           mol_att_src, mol_att_dst, mol_bias, mol_gru_Wih, mol_gru_Whh,
           mol_gru_bih, mol_gru_bhh, lin2_W, lin2_b):
    src, dst = edge_index[0], edge_index[1]
    n = x.shape[0]
    eps = 1e-16

    x1 = _lrelu(_matmul(x, lin1_W.T) + lin1_b)

    W1a = gate_W1[:, :H]
    W1b = gate_W1[:, H:]
    xW1a = _matmul(x1, W1a.T)
    eW1b = _matmul(edge_attr, W1b.T)
    s_r = _matmul(x1, gate_att_r[:, None])[:, 0]

    xj = _lrelu(xW1a[src] + eW1b)
    q = _matmul(xj, gate_att_l[:, None])[:, 0] + s_r[dst]
    logit = _lrelu(q)
    shift = jnp.max(logit)
    ex = jnp.exp(logit - shift)
    denom = jax.ops.segment_sum(ex, dst, n)
    accw = jax.ops.segment_sum(ex[:, None] * xj, dst, n)
    h = _elu(_matmul(accw / (denom + eps)[:, None], gate_W2.T) + gate_bias)
    xc = jax.nn.relu(_gru(h, x1, gru_Wih[0], gru_Whh[0], gru_bih[0], gru_bhh[0]))

    for i in range(atom_W.shape[0]):
        xs = _matmul(xc, atom_W[i].T)
        s_src = _matmul(xs, atom_att_src[i][:, None])[:, 0]
        s_dst = _matmul(xs, atom_att_dst[i][:, None])[:, 0]
        logit = _lrelu(s_src[src] + s_dst[dst])
        shift = _lrelu(jnp.max(s_src) + jnp.max(s_dst))
        ex = jnp.exp(logit - shift)
        denom = jax.ops.segment_sum(ex, dst, n)
        acc = jax.ops.segment_sum(ex[:, None] * xs[src], dst, n)
        h = _elu(acc / (denom + eps)[:, None] + atom_bias[i])
        xc = jax.nn.relu(_gru(h, xc, gru_Wih[i + 1], gru_Whh[i + 1],
                              gru_bih[i + 1], gru_bhh[i + 1]))

    b_oh = (batch[:, None] == jnp.arange(G_GRAPHS, dtype=batch.dtype)[None, :])
    b_oh = b_oh.astype(jnp.float32)
    out = jax.nn.relu(_matmul(b_oh.T, xc))

    xs_mol = _matmul(xc, mol_W.T)
    s_src_mol = _matmul(xs_mol, mol_att_src[:, None])[:, 0]
    for _ in range(2):
        xd = _matmul(out, mol_W.T)
        s_dst = _matmul(xd, mol_att_dst[:, None])[:, 0]
        logit = _lrelu(s_src_mol + s_dst[batch])
        shift = _lrelu(jnp.max(s_src_mol) + jnp.max(s_dst))
        ex = jnp.exp(logit - shift)
        denom = _matmul(b_oh.T, ex[:, None])[:, 0]
        accm = _matmul(b_oh.T, ex[:, None] * xs_mol)
        h = _elu(accm / (denom + eps)[:, None] + mol_bias)
        out = jax.nn.relu(_gru(h, out, mol_gru_Wih, mol_gru_Whh,
                               mol_gru_bih, mol_gru_bhh))

    return _matmul(out, lin2_W.T) + lin2_b
